# baseline (speedup 1.0000x reference)
import functools

import jax
import jax.numpy as jnp
from jax.experimental import pallas as pl
from jax.experimental.pallas import tpu as pltpu

_N = 6
_NBUF = 4


def _tgam_kernel(x_hbm, w_ref, b_ref, o_ref, bufs, ps_ref, sems):
    NBUF, BB, L, C = bufs.shape
    nsteps = pl.num_programs(0)
    B = ps_ref.shape[1] * ps_ref.shape[2]
    ratio = L // _N
    i = pl.program_id(0)
    lookahead = NBUF - 1

    def issue(blk):
        slot = jax.lax.rem(blk, NBUF)
        pltpu.make_async_copy(
            x_hbm.at[pl.ds(blk * BB, BB)], bufs.at[slot], sems.at[slot]
        ).start()

    @pl.when(i == 0)
    def _prime():
        for k in range(lookahead):
            issue(k)

    @pl.when(i + lookahead < nsteps)
    def _prefetch():
        issue(i + lookahead)

    slot = jax.lax.rem(i, NBUF)
    pltpu.make_async_copy(
        x_hbm.at[pl.ds(i * BB, BB)], bufs.at[slot], sems.at[slot]).wait()

    for n in range(_N):
        rows = [
            jnp.sum(bufs[slot, bb, n * ratio:(n + 1) * ratio, :], axis=0,
                    keepdims=True)
            for bb in range(BB)
        ]
        ps_ref[n, i] = jnp.concatenate(rows, axis=0)

    @pl.when(i == nsteps - 1)
    def _finish():
        inv = 1.0 / ratio
        p = [ps_ref[n].reshape(B, C) * inv for n in range(_N)]

        cols = [[None] * _N for _ in range(_N)]
        zero = jnp.zeros((B, 1), jnp.float32)
        for n in range(_N):
            cols[n][n] = zero
            for m in range(n + 1, _N):
                d = p[n] - p[m]
                s = jnp.sum(d * d, axis=-1, keepdims=True)
                cols[n][m] = s
                cols[m][n] = s
        D = jnp.concatenate(
            [cols[n][m] for n in range(_N) for m in range(_N)], axis=1)
        Dt = D.T
        row = [Dt[k:k + 1, :] for k in range(_N * _N)]

        deg = []
        for m in range(_N):
            dm = jnp.zeros((1, B), jnp.float32)
            for n in range(_N):
                r = jnp.zeros((1, B), jnp.float32)
                for j in range(_N):
                    if j == m:
                        continue
                    if j < m:
                        r += (row[n * _N + j] <= row[n * _N + m]).astype(
                            jnp.float32)
                    else:
                        r += (row[n * _N + j] < row[n * _N + m]).astype(
                            jnp.float32)
                dm += (r <= 2.5).astype(jnp.float32)
            deg.append(dm)
        Cmat = jnp.concatenate(deg, axis=0)
        c = Cmat.T * (1.0 / ((3.0 + 1e-6) * _N))

        g = c[:, 0:1] * p[0]
        for m in range(1, _N):
            g = g + c[:, m:m + 1] * p[m]
        mean_pf = p[0]
        for m in range(1, _N):
            mean_pf = mean_pf + p[m]
        mean_pf = mean_pf * (1.0 / _N)
        out = jax.lax.dot_general(
            g, w_ref[...], (((1,), (1,)), ((), ())),
            preferred_element_type=jnp.float32)
        o_ref[...] = out + b_ref[...] + mean_pf


@jax.jit
def kernel(x, W, b):
    B, L, C = x.shape
    BB = 4
    out = pl.pallas_call(
        _tgam_kernel,
        grid=(B // BB,),
        in_specs=[
            pl.BlockSpec(memory_space=pltpu.MemorySpace.HBM),
            pl.BlockSpec((C, C), lambda i: (0, 0)),
            pl.BlockSpec((1, C), lambda i: (0, 0)),
        ],
        out_specs=pl.BlockSpec((B, C), lambda i: (0, 0)),
        out_shape=jax.ShapeDtypeStruct((B, C), x.dtype),
        scratch_shapes=[
            pltpu.VMEM((_NBUF, BB, L, C), jnp.float32),
            pltpu.VMEM((_N, B // BB, BB, C), jnp.float32),
            pltpu.SemaphoreType.DMA((_NBUF,)),
        ],
        compiler_params=pltpu.CompilerParams(
            dimension_semantics=("arbitrary",),
        ),
    )(x, W, b.reshape(1, C))
    return out


# speedup vs baseline: 1.0045x; 1.0045x over previous
import functools

import jax
import jax.numpy as jnp
from jax.experimental import pallas as pl
from jax.experimental.pallas import tpu as pltpu

_N = 6
_NBUF = 6


def _tgam_kernel(x_hbm, w_ref, b_ref, o_ref, bufs, ps_ref, sems):
    NBUF, BB, L, C = bufs.shape
    nsteps = pl.num_programs(0)
    B = ps_ref.shape[1] * ps_ref.shape[2]
    ratio = L // _N
    i = pl.program_id(0)
    lookahead = NBUF - 1

    def issue(blk):
        slot = jax.lax.rem(blk, NBUF)
        pltpu.make_async_copy(
            x_hbm.at[pl.ds(blk * BB, BB)], bufs.at[slot], sems.at[slot]
        ).start()

    @pl.when(i == 0)
    def _prime():
        for k in range(lookahead):
            issue(k)

    @pl.when(i + lookahead < nsteps)
    def _prefetch():
        issue(i + lookahead)

    slot = jax.lax.rem(i, NBUF)
    pltpu.make_async_copy(
        x_hbm.at[pl.ds(i * BB, BB)], bufs.at[slot], sems.at[slot]).wait()

    for n in range(_N):
        rows = [
            jnp.sum(bufs[slot, bb, n * ratio:(n + 1) * ratio, :], axis=0,
                    keepdims=True)
            for bb in range(BB)
        ]
        ps_ref[n, i] = jnp.concatenate(rows, axis=0)

    @pl.when(i == nsteps - 1)
    def _finish():
        inv = 1.0 / ratio
        p = [ps_ref[n].reshape(B, C) * inv for n in range(_N)]

        cols = [[None] * _N for _ in range(_N)]
        zero = jnp.zeros((B, 1), jnp.float32)
        for n in range(_N):
            cols[n][n] = zero
            for m in range(n + 1, _N):
                d = p[n] - p[m]
                s = jnp.sum(d * d, axis=-1, keepdims=True)
                cols[n][m] = s
                cols[m][n] = s
        D = jnp.concatenate(
            [cols[n][m] for n in range(_N) for m in range(_N)], axis=1)
        Dt = D.T
        row = [Dt[k:k + 1, :] for k in range(_N * _N)]

        deg = []
        for m in range(_N):
            dm = jnp.zeros((1, B), jnp.float32)
            for n in range(_N):
                r = jnp.zeros((1, B), jnp.float32)
                for j in range(_N):
                    if j == m:
                        continue
                    if j < m:
                        r += (row[n * _N + j] <= row[n * _N + m]).astype(
                            jnp.float32)
                    else:
                        r += (row[n * _N + j] < row[n * _N + m]).astype(
                            jnp.float32)
                dm += (r <= 2.5).astype(jnp.float32)
            deg.append(dm)
        Cmat = jnp.concatenate(deg, axis=0)
        c = Cmat.T * (1.0 / ((3.0 + 1e-6) * _N))

        g = c[:, 0:1] * p[0]
        for m in range(1, _N):
            g = g + c[:, m:m + 1] * p[m]
        mean_pf = p[0]
        for m in range(1, _N):
            mean_pf = mean_pf + p[m]
        mean_pf = mean_pf * (1.0 / _N)
        out = jax.lax.dot_general(
            g, w_ref[...], (((1,), (1,)), ((), ())),
            preferred_element_type=jnp.float32)
        o_ref[...] = out + b_ref[...] + mean_pf


@jax.jit
def kernel(x, W, b):
    B, L, C = x.shape
    BB = 2
    out = pl.pallas_call(
        _tgam_kernel,
        grid=(B // BB,),
        in_specs=[
            pl.BlockSpec(memory_space=pltpu.MemorySpace.HBM),
            pl.BlockSpec((C, C), lambda i: (0, 0)),
            pl.BlockSpec((1, C), lambda i: (0, 0)),
        ],
        out_specs=pl.BlockSpec((B, C), lambda i: (0, 0)),
        out_shape=jax.ShapeDtypeStruct((B, C), x.dtype),
        scratch_shapes=[
            pltpu.VMEM((_NBUF, BB, L, C), jnp.float32),
            pltpu.VMEM((_N, B // BB, BB, C), jnp.float32),
            pltpu.SemaphoreType.DMA((_NBUF,)),
        ],
        compiler_params=pltpu.CompilerParams(
            dimension_semantics=("arbitrary",),
        ),
    )(x, W, b.reshape(1, C))
    return out


# speedup vs baseline: 1.0188x; 1.0142x over previous
import jax
import jax.numpy as jnp
from jax.experimental import pallas as pl
from jax.experimental.pallas import tpu as pltpu

_N = 6


def _tgam_kernel(x_ref, w_ref, b_ref, o_ref, ps_ref):
    BB, L, C = x_ref.shape
    B = ps_ref.shape[1] * ps_ref.shape[2]
    ratio = L // _N
    i = pl.program_id(0)
    nsteps = pl.num_programs(0)

    for n in range(_N):
        rows = [
            jnp.sum(x_ref[bb, n * ratio:(n + 1) * ratio, :], axis=0,
                    keepdims=True)
            for bb in range(BB)
        ]
        ps_ref[n, i] = jnp.concatenate(rows, axis=0)

    @pl.when(i == nsteps - 1)
    def _finish():
        inv = 1.0 / ratio
        p = [ps_ref[n].reshape(B, C) * inv for n in range(_N)]

        cols = [[None] * _N for _ in range(_N)]
        zero = jnp.zeros((B, 1), jnp.float32)
        for n in range(_N):
            cols[n][n] = zero
            for m in range(n + 1, _N):
                d = p[n] - p[m]
                s = jnp.sum(d * d, axis=-1, keepdims=True)
                cols[n][m] = s
                cols[m][n] = s
        D = jnp.concatenate(
            [cols[n][m] for n in range(_N) for m in range(_N)], axis=1)
        Dt = D.T
        row = [Dt[k:k + 1, :] for k in range(_N * _N)]

        deg = []
        for m in range(_N):
            dm = jnp.zeros((1, B), jnp.float32)
            for n in range(_N):
                r = jnp.zeros((1, B), jnp.float32)
                for j in range(_N):
                    if j == m:
                        continue
                    if j < m:
                        r += (row[n * _N + j] <= row[n * _N + m]).astype(
                            jnp.float32)
                    else:
                        r += (row[n * _N + j] < row[n * _N + m]).astype(
                            jnp.float32)
                dm += (r <= 2.5).astype(jnp.float32)
            deg.append(dm)
        Cmat = jnp.concatenate(deg, axis=0)
        c = Cmat.T * (1.0 / ((3.0 + 1e-6) * _N))

        g = c[:, 0:1] * p[0]
        for m in range(1, _N):
            g = g + c[:, m:m + 1] * p[m]
        mean_pf = p[0]
        for m in range(1, _N):
            mean_pf = mean_pf + p[m]
        mean_pf = mean_pf * (1.0 / _N)
        out = jax.lax.dot_general(
            g, w_ref[...], (((1,), (1,)), ((), ())),
            preferred_element_type=jnp.float32)
        o_ref[...] = out + b_ref[...] + mean_pf


@jax.jit
def kernel(x, W, b):
    B, L, C = x.shape
    BB = 4
    out = pl.pallas_call(
        _tgam_kernel,
        grid=(B // BB,),
        in_specs=[
            pl.BlockSpec((BB, L, C), lambda i: (i, 0, 0)),
            pl.BlockSpec((C, C), lambda i: (0, 0)),
            pl.BlockSpec((1, C), lambda i: (0, 0)),
        ],
        out_specs=pl.BlockSpec((B, C), lambda i: (0, 0)),
        out_shape=jax.ShapeDtypeStruct((B, C), x.dtype),
        scratch_shapes=[pltpu.VMEM((_N, B // BB, BB, C), jnp.float32)],
        compiler_params=pltpu.CompilerParams(
            dimension_semantics=("arbitrary",),
        ),
    )(x, W, b.reshape(1, C))
    return out
